# Initial kernel scaffold; baseline (speedup 1.0000x reference)
import jax
import jax.numpy as jnp
from jax.experimental import pallas as pl


def kernel(x, edge_index, Wq, Wk, Wv, Wo):
    raise NotImplementedError("write your pallas kernel here")


# speedup vs baseline: 18.0183x; 18.0183x over previous
import jax
import jax.numpy as jnp
from jax import lax
from jax.experimental import pallas as pl
from jax.experimental.pallas import tpu as pltpu
from jax.experimental.pallas import tpu_sc as plsc

N = 10000
E = 320000
D = 128
H = 8
DK = 16

NC = 2
NS = 16
NW = NC * NS
EP = E // NW
C = 80
NCHUNK = EP // C
ROW = D + 2 * H
NP = 10240
RPT = NP // NS


def _proj_body(x_ref, w_ref, q_ref, k_ref, v_ref):
    full = jnp.dot(x_ref[...], w_ref[...], preferred_element_type=jnp.float32)
    q_ref[...] = full[:, :D]
    k_ref[...] = full[:, D:2 * D]
    v_ref[...] = full[:, 2 * D:]


def _project(x, w):
    blk = 1000
    return pl.pallas_call(
        _proj_body,
        grid=(N // blk,),
        in_specs=[
            pl.BlockSpec((blk, D), lambda i: (i, 0)),
            pl.BlockSpec((D, 3 * D), lambda i: (0, 0)),
        ],
        out_specs=[pl.BlockSpec((blk, D), lambda i: (i, 0))] * 3,
        out_shape=[jax.ShapeDtypeStruct((N, D), jnp.float32)] * 3,
    )(x, w)


def _edge_body(q_hbm, k_hbm, v_hbm, src_hbm, dst_hbm, part_hbm,
               acc_sp, src_v, dst_v, a_v, b_v, out_v,
               sem_a, sem_b, sem_s):
    core = lax.axis_index("c")
    sub = lax.axis_index("s")
    wid = sub * NC + core

    def _zrow(r, carry):
        for c16 in range(ROW // 16):
            out_v[r, pl.ds(c16 * 16, 16)] = jnp.zeros((16,), jnp.float32)
        return carry
    lax.fori_loop(0, C, _zrow, 0)
    for j in range(RPT // C):
        pltpu.sync_copy(out_v, acc_sp.at[pl.ds(sub * RPT + j * C, C)])

    plsc.subcore_barrier()

    iota = lax.broadcasted_iota(jnp.int32, (16,), 0)

    def _chunk(i, carry):
        base = wid * EP + i * C
        pltpu.sync_copy(src_hbm.at[pl.ds(base, C)], src_v)
        pltpu.sync_copy(dst_hbm.at[pl.ds(base, C)], dst_v)
        cp_k = pltpu.async_copy(k_hbm.at[src_v], a_v, sem_a)
        cp_q = pltpu.async_copy(q_hbm.at[dst_v], b_v, sem_b)
        cp_k.wait()
        cp_q.wait()

        for g in range(C // 16):
            e_vec = iota + g * 16
            for h in range(H):
                def _dot(d, acc):
                    col = jnp.full((16,), h * DK + d, jnp.int32)
                    kc = plsc.load_gather(a_v, [e_vec, col])
                    qc = plsc.load_gather(b_v, [e_vec, col])
                    return acc + kc * qc
                s = lax.fori_loop(0, DK, _dot, jnp.zeros((16,), jnp.float32))
                s = s * 0.25
                s = jnp.minimum(jnp.maximum(s, -10.0), 10.0)
                p = jnp.exp(s)
                plsc.store_scatter(
                    out_v, [e_vec, jnp.full((16,), D + h, jnp.int32)], p)

        pltpu.async_copy(v_hbm.at[src_v], a_v, sem_a).wait()

        def _wv(e, carry):
            for h in range(H):
                sc = plsc.load_gather(
                    out_v, [jnp.full((16,), e, jnp.int32),
                            jnp.full((16,), D + h, jnp.int32)])
                out_v[e, pl.ds(h * DK, DK)] = a_v[e, pl.ds(h * DK, DK)] * sc
            return carry
        lax.fori_loop(0, C, _wv, 0)

        pltpu.async_copy(out_v, acc_sp.at[dst_v], sem_s, add=True).wait()
        return carry

    lax.fori_loop(0, NCHUNK, _chunk, 0)

    plsc.subcore_barrier()

    pltpu.sync_copy(acc_sp.at[pl.ds(sub * RPT, RPT)],
                    part_hbm.at[core, pl.ds(sub * RPT, RPT)])


def _edge_attention(q_tab, k_tab, v_tab, src, dst):
    mesh = plsc.VectorSubcoreMesh(core_axis_name="c", subcore_axis_name="s")
    return pl.kernel(
        _edge_body,
        out_type=jax.ShapeDtypeStruct((NC, NP, ROW), jnp.float32),
        mesh=mesh,
        compiler_params=pltpu.CompilerParams(
            use_tc_tiling_on_sc=False, needs_layout_passes=False),
        scratch_types=[
            pltpu.VMEM_SHARED((NP, ROW), jnp.float32),
            pltpu.VMEM((C,), jnp.int32),
            pltpu.VMEM((C,), jnp.int32),
            pltpu.VMEM((C, D), jnp.float32),
            pltpu.VMEM((C, D), jnp.float32),
            pltpu.VMEM((C, ROW), jnp.float32),
            pltpu.SemaphoreType.DMA,
            pltpu.SemaphoreType.DMA,
            pltpu.SemaphoreType.DMA,
        ],
    )(q_tab, k_tab, v_tab, src, dst)


def _out_body(part_ref, wo_ref, o_ref):
    both = part_ref[...]
    tot = both[0] + both[1]
    wv = tot[:, :D]
    z = tot[:, D:D + H]
    rows = lax.broadcasted_iota(jnp.int32, (H, D), 0)
    cols = lax.broadcasted_iota(jnp.int32, (H, D), 1)
    sel = (cols // DK == rows).astype(jnp.float32)
    norm = jnp.dot(z, sel, preferred_element_type=jnp.float32) + 1e-6
    o_ref[...] = jnp.dot(wv / norm, wo_ref[...],
                         preferred_element_type=jnp.float32)


def _finalize(part, wo):
    blk = 1000
    return pl.pallas_call(
        _out_body,
        grid=(N // blk,),
        in_specs=[
            pl.BlockSpec((NC, blk, ROW), lambda i: (0, i, 0)),
            pl.BlockSpec((D, D), lambda i: (0, 0)),
        ],
        out_specs=pl.BlockSpec((blk, D), lambda i: (i, 0)),
        out_shape=jax.ShapeDtypeStruct((N, D), jnp.float32),
    )(part, wo)


@jax.jit
def kernel(x, edge_index, Wq, Wk, Wv, Wo):
    w = jnp.concatenate([Wq, Wk, Wv], axis=1)
    q_tab, k_tab, v_tab = _project(x, w)
    src = edge_index[0].astype(jnp.int32)
    dst = edge_index[1].astype(jnp.int32)
    part = _edge_attention(q_tab, k_tab, v_tab, src, dst)
    return _finalize(part, Wo)
